# Initial kernel scaffold; baseline (speedup 1.0000x reference)
import functools
import math

import jax
import jax.numpy as jnp
from jax import lax
from jax.experimental import pallas as pl
from jax.experimental.pallas import tpu as pltpu

HID = 128
NH = 4
DH = 32


def _cdiv(a, b):
    return (a + b - 1) // b


def _prep_body(x_ref, wq_ref, bq_ref, sc_ref, wk_ref, bk_ref, abd_ref,
               wv_ref, bv_ref, mbd_ref, q_out, kr_out, vr_out):
    x = x_ref[...]
    q = jnp.dot(x, wq_ref[...], preferred_element_type=jnp.float32) + bq_ref[...]
    q_out[...] = q * sc_ref[...]
    k = jnp.dot(x, wk_ref[...], preferred_element_type=jnp.float32) + bk_ref[...]
    kr_out[...] = jnp.dot(k, abd_ref[...], preferred_element_type=jnp.float32)
    v = jnp.dot(x, wv_ref[...], preferred_element_type=jnp.float32) + bv_ref[...]
    vr_out[...] = jnp.dot(v, mbd_ref[...], preferred_element_type=jnp.float32)


def _prep(x, WqT, bq, scale, WkT, bk, Abd, WvT, bv, Mbd, B=512):
    n = x.shape[0]
    row = pl.BlockSpec((B, HID), lambda i: (i, 0))
    full = pl.BlockSpec((HID, HID), lambda i: (0, 0))
    vec = pl.BlockSpec((1, HID), lambda i: (0, 0))
    return pl.pallas_call(
        _prep_body,
        grid=(_cdiv(n, B),),
        in_specs=[row, full, vec, vec, full, vec, full, full, vec, full],
        out_specs=[row, row, row],
        out_shape=[jax.ShapeDtypeStruct((n, HID), jnp.float32)] * 3,
    )(x, WqT, bq.reshape(1, HID), scale.reshape(1, HID), WkT,
      bk.reshape(1, HID), Abd, WvT, bv.reshape(1, HID), Mbd)


def _ent_a_body(num_ref, den_ref, x_ref, waT_ref, ba_ref, sk_ref,
                inpT_ref, binp_ref, h_out, qkv_out):
    msg = num_ref[...] / (den_ref[...] + 1e-16)
    g = 0.5 * msg * (1.0 + lax.erf(msg * (2.0 ** -0.5)))
    o = jnp.dot(g, waT_ref[...], preferred_element_type=jnp.float32) + ba_ref[...]
    be = 1.0 / (1.0 + jnp.exp(-sk_ref[0, 0]))
    h = be * o + (1.0 - be) * x_ref[...]
    h_out[...] = h
    qkv_out[...] = jnp.dot(h, inpT_ref[...], preferred_element_type=jnp.float32) + binp_ref[...]


def _ent_a(num, den, x, WaT, ba, skip, inpT, binp, B=512):
    n = x.shape[0]
    row = pl.BlockSpec((B, HID), lambda i: (i, 0))
    return pl.pallas_call(
        _ent_a_body,
        grid=(_cdiv(n, B),),
        in_specs=[row, row, row,
                  pl.BlockSpec((HID, HID), lambda i: (0, 0)),
                  pl.BlockSpec((1, HID), lambda i: (0, 0)),
                  pl.BlockSpec((1, 1), lambda i: (0, 0)),
                  pl.BlockSpec((HID, 3 * HID), lambda i: (0, 0)),
                  pl.BlockSpec((1, 3 * HID), lambda i: (0, 0))],
        out_specs=[row, pl.BlockSpec((B, 3 * HID), lambda i: (i, 0))],
        out_shape=[jax.ShapeDtypeStruct((n, HID), jnp.float32),
                   jax.ShapeDtypeStruct((n, 3 * HID), jnp.float32)],
    )(num, den, x, WaT, ba.reshape(1, HID), skip.reshape(1, 1), inpT,
      binp.reshape(1, 3 * HID))


def _att_body(q_ref, k_ref, v_ref, o_ref):
    q = q_ref[0]
    k = k_ref[0]
    s = lax.dot_general(q, k, (((1,), (1,)), ((), ())),
                        preferred_element_type=jnp.float32) * (DH ** -0.5)
    m = jnp.max(s, axis=-1, keepdims=True)
    p = jnp.exp(s - m)
    p = p / jnp.sum(p, axis=-1, keepdims=True)
    o_ref[0] = jnp.dot(p, v_ref[0], preferred_element_type=jnp.float32)


def _att(qh, kh, vh, B=1024):
    n = qh.shape[1]
    return pl.pallas_call(
        _att_body,
        grid=(NH, _cdiv(n, B)),
        in_specs=[pl.BlockSpec((1, B, DH), lambda h, i: (h, i, 0)),
                  pl.BlockSpec((1, n, DH), lambda h, i: (h, 0, 0)),
                  pl.BlockSpec((1, n, DH), lambda h, i: (h, 0, 0))],
        out_specs=pl.BlockSpec((1, B, DH), lambda h, i: (h, i, 0)),
        out_shape=jax.ShapeDtypeStruct((NH, n, DH), jnp.float32),
    )(qh, kh, vh)


def _ent_c_body(h_ref, g4_ref, outT_ref, bout_ref, g_ref, b_ref, qe_ref, w_out):
    glob = jnp.dot(g4_ref[...], outT_ref[...],
                   preferred_element_type=jnp.float32) + bout_ref[...]
    t = 0.9 * h_ref[...] + 0.1 * glob
    mu = jnp.mean(t, axis=-1, keepdims=True)
    var = jnp.mean((t - mu) ** 2, axis=-1, keepdims=True)
    h2 = (t - mu) / jnp.sqrt(var + 1e-5) * g_ref[...] + b_ref[...]
    r = jnp.sum(h2 * qe_ref[...], axis=-1, keepdims=True)
    rel = 1.0 / (1.0 + jnp.exp(-r))
    w_out[...] = h2 * rel


def _ent_c(h, g4, outT, bout, g, b, qe, B=512):
    n = h.shape[0]
    row = pl.BlockSpec((B, HID), lambda i: (i, 0))
    vec = pl.BlockSpec((1, HID), lambda i: (0, 0))
    return pl.pallas_call(
        _ent_c_body,
        grid=(_cdiv(n, B),),
        in_specs=[row, row, pl.BlockSpec((HID, HID), lambda i: (0, 0)),
                  vec, vec, vec, vec],
        out_specs=row,
        out_shape=jax.ShapeDtypeStruct((n, HID), jnp.float32),
    )(h, g4, outT, bout.reshape(1, HID), g.reshape(1, HID),
      b.reshape(1, HID), qe.reshape(1, HID))


def _psg_body(num_ref, den_ref, x_ref, ctx_ref, waT_ref, ba_ref, sk_ref,
              g_ref, b_ref, w1aT_ref, w1bT_ref, b1_ref, qe_ref, w2T_ref,
              b2_ref, s_out):
    msg = num_ref[...] / (den_ref[...] + 1e-16)
    gl = 0.5 * msg * (1.0 + lax.erf(msg * (2.0 ** -0.5)))
    o = jnp.dot(gl, waT_ref[...], preferred_element_type=jnp.float32) + ba_ref[...]
    bp = 1.0 / (1.0 + jnp.exp(-sk_ref[0, 0]))
    h = bp * o + (1.0 - bp) * x_ref[...]
    t = h + ctx_ref[...]
    mu = jnp.mean(t, axis=-1, keepdims=True)
    var = jnp.mean((t - mu) ** 2, axis=-1, keepdims=True)
    h2 = (t - mu) / jnp.sqrt(var + 1e-5) * g_ref[...] + b_ref[...]
    qc = jnp.dot(qe_ref[...], w1bT_ref[...], preferred_element_type=jnp.float32)
    hid = jnp.maximum(
        jnp.dot(h2, w1aT_ref[...], preferred_element_type=jnp.float32)
        + qc + b1_ref[...], 0.0)
    s_out[...] = jnp.dot(hid, w2T_ref[...],
                         preferred_element_type=jnp.float32) + b2_ref[...]


def _psg(num, den, x, ctx, WaT, ba, skip, g, b, W1aT, W1bT, b1, qe, W2T, b2,
         B=512):
    n = x.shape[0]
    row = pl.BlockSpec((B, HID), lambda i: (i, 0))
    vec = pl.BlockSpec((1, HID), lambda i: (0, 0))
    full = pl.BlockSpec((HID, HID), lambda i: (0, 0))
    return pl.pallas_call(
        _psg_body,
        grid=(_cdiv(n, B),),
        in_specs=[row, row, row, row, full, vec,
                  pl.BlockSpec((1, 1), lambda i: (0, 0)),
                  vec, vec, full, full, vec, vec,
                  pl.BlockSpec((HID, 8), lambda i: (0, 0)),
                  pl.BlockSpec((1, 8), lambda i: (0, 0))],
        out_specs=pl.BlockSpec((B, 8), lambda i: (i, 0)),
        out_shape=jax.ShapeDtypeStruct((n, 8), jnp.float32),
    )(num, den, x, ctx, WaT, ba.reshape(1, HID), skip.reshape(1, 1),
      g.reshape(1, HID), b.reshape(1, HID), W1aT, W1bT, b1.reshape(1, HID),
      qe.reshape(1, HID), W2T, b2.reshape(1, 8))


def _edges_jax(q, kr, vr, src, dst, n_dst):
    al = jnp.sum(q[dst].reshape(-1, NH, DH) * kr[src].reshape(-1, NH, DH), -1)
    ex = jnp.exp(al)
    num = jax.ops.segment_sum(
        (vr[src].reshape(-1, NH, DH) * ex[:, :, None]).reshape(-1, HID),
        dst, num_segments=n_dst)
    den = jax.ops.segment_sum(ex, dst, num_segments=n_dst)
    return num, den


def kernel(x_entity, x_passage, edge_index_e2p, edge_index_p2e, query_emb,
           params):
    p = params
    n_ent = x_entity.shape[0]
    n_psg = x_passage.shape[0]

    def bd(a):
        return jax.scipy.linalg.block_diag(*[a[h] for h in range(NH)])

    isq = 1.0 / math.sqrt(DH)
    sc_e = jnp.repeat(p['p_rel_p2e'], DH) * isq
    sc_p = jnp.repeat(p['p_rel_e2p'], DH) * isq

    q_e, kr_e2p, vr_e2p = _prep(
        x_entity, p['Wq_ent'].T, p['bq_ent'], sc_e, p['Wk_ent'].T,
        p['bk_ent'], bd(p['a_rel_e2p']), p['Wv_ent'].T, p['bv_ent'],
        bd(p['m_rel_e2p']))
    q_p, kr_p2e, vr_p2e = _prep(
        x_passage, p['Wq_psg'].T, p['bq_psg'], sc_p, p['Wk_psg'].T,
        p['bk_psg'], bd(p['a_rel_p2e']), p['Wv_psg'].T, p['bv_psg'],
        bd(p['m_rel_p2e']))

    s_ep, d_ep = edge_index_e2p[0], edge_index_e2p[1]
    s_pe, d_pe = edge_index_p2e[0], edge_index_p2e[1]
    num_p, den_p = _edges_jax(q_p, kr_e2p, vr_e2p, s_ep, d_ep, n_psg)
    num_e, den_e = _edges_jax(q_e, kr_p2e, vr_p2e, s_pe, d_pe, n_ent)
    den_p128 = jnp.repeat(den_p, DH, axis=-1)
    den_e128 = jnp.repeat(den_e, DH, axis=-1)

    h_ent, qkv = _ent_a(num_e, den_e128, x_entity, p['Wa_ent'].T,
                        p['ba_ent'], p['skip_ent'], p['in_proj_w'].T,
                        p['in_proj_b'])
    qh = qkv[:, :HID].reshape(n_ent, NH, DH).transpose(1, 0, 2)
    kh = qkv[:, HID:2 * HID].reshape(n_ent, NH, DH).transpose(1, 0, 2)
    vh = qkv[:, 2 * HID:].reshape(n_ent, NH, DH).transpose(1, 0, 2)
    g4 = _att(qh, kh, vh).transpose(1, 0, 2).reshape(n_ent, HID)
    weighted = _ent_c(h_ent, g4, p['out_proj_w'].T, p['out_proj_b'],
                      p['g_ent'], p['be_ent'], query_emb)

    ctx = jax.ops.segment_sum(weighted[s_ep], d_ep, num_segments=n_psg)

    W1aT = p['W1'][:, :HID].T
    W1bT = p['W1'][:, HID:].T
    W2T = jnp.zeros((HID, 8), jnp.float32).at[:, 0].set(p['W2'][0])
    b2 = jnp.zeros((8,), jnp.float32).at[0].set(p['b2'][0])
    scores = _psg(num_p, den_p128, x_passage, ctx, p['Wa_psg'].T,
                  p['ba_psg'], p['skip_psg'], p['g_psg'], p['be_psg'],
                  W1aT, W1bT, p['b1'], query_emb, W2T, b2)
    return scores[:, 0]


# speedup vs baseline: 9.5176x; 9.5176x over previous
import functools
import math

import jax
import jax.numpy as jnp
from jax import lax
from jax.experimental import pallas as pl
from jax.experimental.pallas import tpu as pltpu

HID = 128
NH = 4
DH = 32


def _cdiv(a, b):
    return (a + b - 1) // b


def _prep_body(x_ref, wq_ref, bq_ref, sc_ref, wk_ref, bk_ref, abd_ref,
               wv_ref, bv_ref, mbd_ref, q_out, kr_out, vr_out):
    x = x_ref[...]
    q = jnp.dot(x, wq_ref[...], preferred_element_type=jnp.float32, precision=lax.Precision.HIGHEST) + bq_ref[...]
    q_out[...] = q * sc_ref[...]
    k = jnp.dot(x, wk_ref[...], preferred_element_type=jnp.float32, precision=lax.Precision.HIGHEST) + bk_ref[...]
    kr_out[...] = jnp.dot(k, abd_ref[...], preferred_element_type=jnp.float32, precision=lax.Precision.HIGHEST)
    v = jnp.dot(x, wv_ref[...], preferred_element_type=jnp.float32, precision=lax.Precision.HIGHEST) + bv_ref[...]
    vr_out[...] = jnp.dot(v, mbd_ref[...], preferred_element_type=jnp.float32, precision=lax.Precision.HIGHEST)


def _prep(x, WqT, bq, scale, WkT, bk, Abd, WvT, bv, Mbd, B=512):
    n = x.shape[0]
    row = pl.BlockSpec((B, HID), lambda i: (i, 0))
    full = pl.BlockSpec((HID, HID), lambda i: (0, 0))
    vec = pl.BlockSpec((1, HID), lambda i: (0, 0))
    return pl.pallas_call(
        _prep_body,
        grid=(_cdiv(n, B),),
        in_specs=[row, full, vec, vec, full, vec, full, full, vec, full],
        out_specs=[row, row, row],
        out_shape=[jax.ShapeDtypeStruct((n, HID), jnp.float32)] * 3,
    )(x, WqT, bq.reshape(1, HID), scale.reshape(1, HID), WkT,
      bk.reshape(1, HID), Abd, WvT, bv.reshape(1, HID), Mbd)


def _ent_a_body(num_ref, den_ref, x_ref, waT_ref, ba_ref, sk_ref,
                inpT_ref, binp_ref, h_out, qkv_out):
    msg = num_ref[...] / (den_ref[...] + 1e-16)
    g = 0.5 * msg * (1.0 + lax.erf(msg * (2.0 ** -0.5)))
    o = jnp.dot(g, waT_ref[...], preferred_element_type=jnp.float32, precision=lax.Precision.HIGHEST) + ba_ref[...]
    be = 1.0 / (1.0 + jnp.exp(-sk_ref[0, 0]))
    h = be * o + (1.0 - be) * x_ref[...]
    h_out[...] = h
    qkv_out[...] = jnp.dot(h, inpT_ref[...], preferred_element_type=jnp.float32, precision=lax.Precision.HIGHEST) + binp_ref[...]


def _ent_a(num, den, x, WaT, ba, skip, inpT, binp, B=512):
    n = x.shape[0]
    row = pl.BlockSpec((B, HID), lambda i: (i, 0))
    return pl.pallas_call(
        _ent_a_body,
        grid=(_cdiv(n, B),),
        in_specs=[row, row, row,
                  pl.BlockSpec((HID, HID), lambda i: (0, 0)),
                  pl.BlockSpec((1, HID), lambda i: (0, 0)),
                  pl.BlockSpec((1, 1), lambda i: (0, 0)),
                  pl.BlockSpec((HID, 3 * HID), lambda i: (0, 0)),
                  pl.BlockSpec((1, 3 * HID), lambda i: (0, 0))],
        out_specs=[row, pl.BlockSpec((B, 3 * HID), lambda i: (i, 0))],
        out_shape=[jax.ShapeDtypeStruct((n, HID), jnp.float32),
                   jax.ShapeDtypeStruct((n, 3 * HID), jnp.float32)],
    )(num, den, x, WaT, ba.reshape(1, HID), skip.reshape(1, 1), inpT,
      binp.reshape(1, 3 * HID))


def _att_body(q_ref, k_ref, v_ref, o_ref):
    q = q_ref[0]
    k = k_ref[0]
    s = lax.dot_general(q, k, (((1,), (1,)), ((), ())),
                        preferred_element_type=jnp.float32,
                        precision=lax.Precision.HIGHEST) * (DH ** -0.5)
    m = jnp.max(s, axis=-1, keepdims=True)
    p = jnp.exp(s - m)
    p = p / jnp.sum(p, axis=-1, keepdims=True)
    o_ref[0] = jnp.dot(p, v_ref[0], preferred_element_type=jnp.float32, precision=lax.Precision.HIGHEST)


def _att(qh, kh, vh, B=1024):
    n = qh.shape[1]
    return pl.pallas_call(
        _att_body,
        grid=(NH, _cdiv(n, B)),
        in_specs=[pl.BlockSpec((1, B, DH), lambda h, i: (h, i, 0)),
                  pl.BlockSpec((1, n, DH), lambda h, i: (h, 0, 0)),
                  pl.BlockSpec((1, n, DH), lambda h, i: (h, 0, 0))],
        out_specs=pl.BlockSpec((1, B, DH), lambda h, i: (h, i, 0)),
        out_shape=jax.ShapeDtypeStruct((NH, n, DH), jnp.float32),
    )(qh, kh, vh)


def _ent_c_body(h_ref, g4_ref, outT_ref, bout_ref, g_ref, b_ref, qe_ref, w_out):
    glob = jnp.dot(g4_ref[...], outT_ref[...],
                   preferred_element_type=jnp.float32, precision=lax.Precision.HIGHEST) + bout_ref[...]
    t = 0.9 * h_ref[...] + 0.1 * glob
    mu = jnp.mean(t, axis=-1, keepdims=True)
    var = jnp.mean((t - mu) ** 2, axis=-1, keepdims=True)
    h2 = (t - mu) / jnp.sqrt(var + 1e-5) * g_ref[...] + b_ref[...]
    r = jnp.sum(h2 * qe_ref[...], axis=-1, keepdims=True)
    rel = 1.0 / (1.0 + jnp.exp(-r))
    w_out[...] = h2 * rel


def _ent_c(h, g4, outT, bout, g, b, qe, B=512):
    n = h.shape[0]
    row = pl.BlockSpec((B, HID), lambda i: (i, 0))
    vec = pl.BlockSpec((1, HID), lambda i: (0, 0))
    return pl.pallas_call(
        _ent_c_body,
        grid=(_cdiv(n, B),),
        in_specs=[row, row, pl.BlockSpec((HID, HID), lambda i: (0, 0)),
                  vec, vec, vec, vec],
        out_specs=row,
        out_shape=jax.ShapeDtypeStruct((n, HID), jnp.float32),
    )(h, g4, outT, bout.reshape(1, HID), g.reshape(1, HID),
      b.reshape(1, HID), qe.reshape(1, HID))


def _psg_body(num_ref, den_ref, x_ref, ctx_ref, waT_ref, ba_ref, sk_ref,
              g_ref, b_ref, w1aT_ref, w1bT_ref, b1_ref, qe_ref, w2T_ref,
              b2_ref, s_out):
    msg = num_ref[...] / (den_ref[...] + 1e-16)
    gl = 0.5 * msg * (1.0 + lax.erf(msg * (2.0 ** -0.5)))
    o = jnp.dot(gl, waT_ref[...], preferred_element_type=jnp.float32, precision=lax.Precision.HIGHEST) + ba_ref[...]
    bp = 1.0 / (1.0 + jnp.exp(-sk_ref[0, 0]))
    h = bp * o + (1.0 - bp) * x_ref[...]
    t = h + ctx_ref[...]
    mu = jnp.mean(t, axis=-1, keepdims=True)
    var = jnp.mean((t - mu) ** 2, axis=-1, keepdims=True)
    h2 = (t - mu) / jnp.sqrt(var + 1e-5) * g_ref[...] + b_ref[...]
    qc = jnp.dot(qe_ref[...], w1bT_ref[...], preferred_element_type=jnp.float32, precision=lax.Precision.HIGHEST)
    hid = jnp.maximum(
        jnp.dot(h2, w1aT_ref[...], preferred_element_type=jnp.float32, precision=lax.Precision.HIGHEST)
        + qc + b1_ref[...], 0.0)
    s_out[...] = jnp.dot(hid, w2T_ref[...],
                         preferred_element_type=jnp.float32, precision=lax.Precision.HIGHEST) + b2_ref[...]


def _psg(num, den, x, ctx, WaT, ba, skip, g, b, W1aT, W1bT, b1, qe, W2T, b2,
         B=512):
    n = x.shape[0]
    row = pl.BlockSpec((B, HID), lambda i: (i, 0))
    vec = pl.BlockSpec((1, HID), lambda i: (0, 0))
    full = pl.BlockSpec((HID, HID), lambda i: (0, 0))
    return pl.pallas_call(
        _psg_body,
        grid=(_cdiv(n, B),),
        in_specs=[row, row, row, row, full, vec,
                  pl.BlockSpec((1, 1), lambda i: (0, 0)),
                  vec, vec, full, full, vec, vec,
                  pl.BlockSpec((HID, 8), lambda i: (0, 0)),
                  pl.BlockSpec((1, 8), lambda i: (0, 0))],
        out_specs=pl.BlockSpec((B, 8), lambda i: (i, 0)),
        out_shape=jax.ShapeDtypeStruct((n, 8), jnp.float32),
    )(num, den, x, ctx, WaT, ba.reshape(1, HID), skip.reshape(1, 1),
      g.reshape(1, HID), b.reshape(1, HID), W1aT, W1bT, b1.reshape(1, HID),
      qe.reshape(1, HID), W2T, b2.reshape(1, 8))


def _edges_jax(q, kr, vr, src, dst, n_dst):
    al = jnp.sum(q[dst].reshape(-1, NH, DH) * kr[src].reshape(-1, NH, DH), -1)
    ex = jnp.exp(al)
    num = jax.ops.segment_sum(
        (vr[src].reshape(-1, NH, DH) * ex[:, :, None]).reshape(-1, HID),
        dst, num_segments=n_dst)
    den = jax.ops.segment_sum(ex, dst, num_segments=n_dst)
    return num, den


def kernel(x_entity, x_passage, edge_index_e2p, edge_index_p2e, query_emb,
           params):
    p = params
    n_ent = x_entity.shape[0]
    n_psg = x_passage.shape[0]

    def bd(a):
        return jax.scipy.linalg.block_diag(*[a[h] for h in range(NH)])

    isq = 1.0 / math.sqrt(DH)
    sc_e = jnp.repeat(p['p_rel_p2e'], DH) * isq
    sc_p = jnp.repeat(p['p_rel_e2p'], DH) * isq

    q_e, kr_e2p, vr_e2p = _prep(
        x_entity, p['Wq_ent'].T, p['bq_ent'], sc_e, p['Wk_ent'].T,
        p['bk_ent'], bd(p['a_rel_e2p']), p['Wv_ent'].T, p['bv_ent'],
        bd(p['m_rel_e2p']))
    q_p, kr_p2e, vr_p2e = _prep(
        x_passage, p['Wq_psg'].T, p['bq_psg'], sc_p, p['Wk_psg'].T,
        p['bk_psg'], bd(p['a_rel_p2e']), p['Wv_psg'].T, p['bv_psg'],
        bd(p['m_rel_p2e']))

    s_ep, d_ep = edge_index_e2p[0], edge_index_e2p[1]
    s_pe, d_pe = edge_index_p2e[0], edge_index_p2e[1]
    num_p, den_p = _edges_jax(q_p, kr_e2p, vr_e2p, s_ep, d_ep, n_psg)
    num_e, den_e = _edges_jax(q_e, kr_p2e, vr_p2e, s_pe, d_pe, n_ent)
    den_p128 = jnp.repeat(den_p, DH, axis=-1)
    den_e128 = jnp.repeat(den_e, DH, axis=-1)

    h_ent, qkv = _ent_a(num_e, den_e128, x_entity, p['Wa_ent'].T,
                        p['ba_ent'], p['skip_ent'], p['in_proj_w'].T,
                        p['in_proj_b'])
    qh = qkv[:, :HID].reshape(n_ent, NH, DH).transpose(1, 0, 2)
    kh = qkv[:, HID:2 * HID].reshape(n_ent, NH, DH).transpose(1, 0, 2)
    vh = qkv[:, 2 * HID:].reshape(n_ent, NH, DH).transpose(1, 0, 2)
    g4 = _att(qh, kh, vh).transpose(1, 0, 2).reshape(n_ent, HID)
    weighted = _ent_c(h_ent, g4, p['out_proj_w'].T, p['out_proj_b'],
                      p['g_ent'], p['be_ent'], query_emb)

    ctx = jax.ops.segment_sum(weighted[s_ep], d_ep, num_segments=n_psg)

    W1aT = p['W1'][:, :HID].T
    W1bT = p['W1'][:, HID:].T
    W2T = jnp.zeros((HID, 8), jnp.float32).at[:, 0].set(p['W2'][0])
    b2 = jnp.zeros((8,), jnp.float32).at[0].set(p['b2'][0])
    scores = _psg(num_p, den_p128, x_passage, ctx, p['Wa_psg'].T,
                  p['ba_psg'], p['skip_psg'], p['g_psg'], p['be_psg'],
                  W1aT, W1bT, p['b1'], query_emb, W2T, b2)
    return scores[:, 0]


# speedup vs baseline: 10.0272x; 1.0535x over previous
import functools
import math

import jax
import jax.numpy as jnp
from jax import lax
from jax.experimental import pallas as pl
from jax.experimental.pallas import tpu as pltpu
from jax.experimental.pallas import tpu_sc as plsc

HID = 128
NH = 4
DH = 32


def _cdiv(a, b):
    return (a + b - 1) // b


def _prep_body(x_ref, wq_ref, bq_ref, sc_ref, wk_ref, bk_ref, abd_ref,
               wv_ref, bv_ref, mbd_ref, q_out, kr_out, vr_out):
    x = x_ref[...]
    q = jnp.dot(x, wq_ref[...], preferred_element_type=jnp.float32, precision=lax.Precision.HIGHEST) + bq_ref[...]
    q_out[...] = q * sc_ref[...]
    k = jnp.dot(x, wk_ref[...], preferred_element_type=jnp.float32, precision=lax.Precision.HIGHEST) + bk_ref[...]
    kr_out[...] = jnp.dot(k, abd_ref[...], preferred_element_type=jnp.float32, precision=lax.Precision.HIGHEST)
    v = jnp.dot(x, wv_ref[...], preferred_element_type=jnp.float32, precision=lax.Precision.HIGHEST) + bv_ref[...]
    vr_out[...] = jnp.dot(v, mbd_ref[...], preferred_element_type=jnp.float32, precision=lax.Precision.HIGHEST)


def _prep(x, WqT, bq, scale, WkT, bk, Abd, WvT, bv, Mbd, B=512):
    n = x.shape[0]
    row = pl.BlockSpec((B, HID), lambda i: (i, 0))
    full = pl.BlockSpec((HID, HID), lambda i: (0, 0))
    vec = pl.BlockSpec((1, HID), lambda i: (0, 0))
    return pl.pallas_call(
        _prep_body,
        grid=(_cdiv(n, B),),
        in_specs=[row, full, vec, vec, full, vec, full, full, vec, full],
        out_specs=[row, row, row],
        out_shape=[jax.ShapeDtypeStruct((n, HID), jnp.float32)] * 3,
    )(x, WqT, bq.reshape(1, HID), scale.reshape(1, HID), WkT,
      bk.reshape(1, HID), Abd, WvT, bv.reshape(1, HID), Mbd)


def _ent_a_body(num_ref, den_ref, x_ref, waT_ref, ba_ref, sk_ref,
                inpT_ref, binp_ref, h_out, qkv_out):
    msg = num_ref[...] / (den_ref[...] + 1e-16)
    g = 0.5 * msg * (1.0 + lax.erf(msg * (2.0 ** -0.5)))
    o = jnp.dot(g, waT_ref[...], preferred_element_type=jnp.float32, precision=lax.Precision.HIGHEST) + ba_ref[...]
    be = 1.0 / (1.0 + jnp.exp(-sk_ref[0, 0]))
    h = be * o + (1.0 - be) * x_ref[...]
    h_out[...] = h
    qkv_out[...] = jnp.dot(h, inpT_ref[...], preferred_element_type=jnp.float32, precision=lax.Precision.HIGHEST) + binp_ref[...]


def _ent_a(num, den, x, WaT, ba, skip, inpT, binp, B=512):
    n = x.shape[0]
    row = pl.BlockSpec((B, HID), lambda i: (i, 0))
    return pl.pallas_call(
        _ent_a_body,
        grid=(_cdiv(n, B),),
        in_specs=[row, row, row,
                  pl.BlockSpec((HID, HID), lambda i: (0, 0)),
                  pl.BlockSpec((1, HID), lambda i: (0, 0)),
                  pl.BlockSpec((1, 1), lambda i: (0, 0)),
                  pl.BlockSpec((HID, 3 * HID), lambda i: (0, 0)),
                  pl.BlockSpec((1, 3 * HID), lambda i: (0, 0))],
        out_specs=[row, pl.BlockSpec((B, 3 * HID), lambda i: (i, 0))],
        out_shape=[jax.ShapeDtypeStruct((n, HID), jnp.float32),
                   jax.ShapeDtypeStruct((n, 3 * HID), jnp.float32)],
    )(num, den, x, WaT, ba.reshape(1, HID), skip.reshape(1, 1), inpT,
      binp.reshape(1, 3 * HID))


def _att_body(q_ref, k_ref, v_ref, o_ref):
    q = q_ref[0]
    k = k_ref[0]
    s = lax.dot_general(q, k, (((1,), (1,)), ((), ())),
                        preferred_element_type=jnp.float32,
                        precision=lax.Precision.HIGHEST) * (DH ** -0.5)
    m = jnp.max(s, axis=-1, keepdims=True)
    p = jnp.exp(s - m)
    p = p / jnp.sum(p, axis=-1, keepdims=True)
    o_ref[0] = jnp.dot(p, v_ref[0], preferred_element_type=jnp.float32, precision=lax.Precision.HIGHEST)


def _att(qh, kh, vh, B=1024):
    n = qh.shape[1]
    return pl.pallas_call(
        _att_body,
        grid=(NH, _cdiv(n, B)),
        in_specs=[pl.BlockSpec((1, B, DH), lambda h, i: (h, i, 0)),
                  pl.BlockSpec((1, n, DH), lambda h, i: (h, 0, 0)),
                  pl.BlockSpec((1, n, DH), lambda h, i: (h, 0, 0))],
        out_specs=pl.BlockSpec((1, B, DH), lambda h, i: (h, i, 0)),
        out_shape=jax.ShapeDtypeStruct((NH, n, DH), jnp.float32),
    )(qh, kh, vh)


def _ent_c_body(h_ref, g4_ref, outT_ref, bout_ref, g_ref, b_ref, qe_ref, w_out):
    glob = jnp.dot(g4_ref[...], outT_ref[...],
                   preferred_element_type=jnp.float32, precision=lax.Precision.HIGHEST) + bout_ref[...]
    t = 0.9 * h_ref[...] + 0.1 * glob
    mu = jnp.mean(t, axis=-1, keepdims=True)
    var = jnp.mean((t - mu) ** 2, axis=-1, keepdims=True)
    h2 = (t - mu) / jnp.sqrt(var + 1e-5) * g_ref[...] + b_ref[...]
    r = jnp.sum(h2 * qe_ref[...], axis=-1, keepdims=True)
    rel = 1.0 / (1.0 + jnp.exp(-r))
    w_out[...] = h2 * rel


def _ent_c(h, g4, outT, bout, g, b, qe, B=512):
    n = h.shape[0]
    row = pl.BlockSpec((B, HID), lambda i: (i, 0))
    vec = pl.BlockSpec((1, HID), lambda i: (0, 0))
    return pl.pallas_call(
        _ent_c_body,
        grid=(_cdiv(n, B),),
        in_specs=[row, row, pl.BlockSpec((HID, HID), lambda i: (0, 0)),
                  vec, vec, vec, vec],
        out_specs=row,
        out_shape=jax.ShapeDtypeStruct((n, HID), jnp.float32),
    )(h, g4, outT, bout.reshape(1, HID), g.reshape(1, HID),
      b.reshape(1, HID), qe.reshape(1, HID))


def _psg_body(num_ref, den_ref, x_ref, ctx_ref, waT_ref, ba_ref, sk_ref,
              g_ref, b_ref, w1aT_ref, w1bT_ref, b1_ref, qe_ref, w2T_ref,
              b2_ref, s_out):
    msg = num_ref[...] / (den_ref[...] + 1e-16)
    gl = 0.5 * msg * (1.0 + lax.erf(msg * (2.0 ** -0.5)))
    o = jnp.dot(gl, waT_ref[...], preferred_element_type=jnp.float32, precision=lax.Precision.HIGHEST) + ba_ref[...]
    bp = 1.0 / (1.0 + jnp.exp(-sk_ref[0, 0]))
    h = bp * o + (1.0 - bp) * x_ref[...]
    t = h + ctx_ref[...]
    mu = jnp.mean(t, axis=-1, keepdims=True)
    var = jnp.mean((t - mu) ** 2, axis=-1, keepdims=True)
    h2 = (t - mu) / jnp.sqrt(var + 1e-5) * g_ref[...] + b_ref[...]
    qc = jnp.dot(qe_ref[...], w1bT_ref[...], preferred_element_type=jnp.float32, precision=lax.Precision.HIGHEST)
    hid = jnp.maximum(
        jnp.dot(h2, w1aT_ref[...], preferred_element_type=jnp.float32, precision=lax.Precision.HIGHEST)
        + qc + b1_ref[...], 0.0)
    s_out[...] = jnp.dot(hid, w2T_ref[...],
                         preferred_element_type=jnp.float32, precision=lax.Precision.HIGHEST) + b2_ref[...]


def _psg(num, den, x, ctx, WaT, ba, skip, g, b, W1aT, W1bT, b1, qe, W2T, b2,
         B=512):
    n = x.shape[0]
    row = pl.BlockSpec((B, HID), lambda i: (i, 0))
    vec = pl.BlockSpec((1, HID), lambda i: (0, 0))
    full = pl.BlockSpec((HID, HID), lambda i: (0, 0))
    return pl.pallas_call(
        _psg_body,
        grid=(_cdiv(n, B),),
        in_specs=[row, row, row, row, full, vec,
                  pl.BlockSpec((1, 1), lambda i: (0, 0)),
                  vec, vec, full, full, vec, vec,
                  pl.BlockSpec((HID, 8), lambda i: (0, 0)),
                  pl.BlockSpec((1, 8), lambda i: (0, 0))],
        out_specs=pl.BlockSpec((B, 8), lambda i: (i, 0)),
        out_shape=jax.ShapeDtypeStruct((n, 8), jnp.float32),
    )(num, den, x, ctx, WaT, ba.reshape(1, HID), skip.reshape(1, 1),
      g.reshape(1, HID), b.reshape(1, HID), W1aT, W1bT, b1.reshape(1, HID),
      qe.reshape(1, HID), W2T, b2.reshape(1, 8))


_CHGT = 128
_CCTX = 256


def _sc_hgt(src, dst, q_tab, k_tab, v_tab, n_dst, split_dst):
    E = src.shape[0]
    if split_dst:
        H = n_dst // 2
        per_tile = E // 16
        C = 64
    else:
        H = n_dst
        per_tile = E // 32
        C = 128
    Hp = _cdiv(H + 16, 128) * 128
    rows_pt = Hp // 16
    trash = _cdiv(H, 32) * 32
    DR = _cdiv(_cdiv(H, 32) + 2, 128) * 128
    drows_pt = DR // 16
    n_chunks = per_tile // C
    mesh = plsc.VectorSubcoreMesh(core_axis_name="c", subcore_axis_name="s")

    @functools.partial(
        pl.kernel, mesh=mesh,
        compiler_params=pltpu.CompilerParams(needs_layout_passes=False),
        out_type=[jax.ShapeDtypeStruct((2, Hp, HID), jnp.float32),
                  jax.ShapeDtypeStruct((2, DR, HID), jnp.float32)],
        scratch_types=[
            pltpu.VMEM_SHARED((Hp, HID), jnp.float32),
            pltpu.VMEM_SHARED((DR, HID), jnp.float32),
            pltpu.VMEM((C,), jnp.int32),
            pltpu.VMEM((C,), jnp.int32),
            pltpu.VMEM((C,), jnp.int32),
            pltpu.VMEM((C,), jnp.int32),
            pltpu.VMEM((C, HID), jnp.float32),
            pltpu.VMEM((C, HID), jnp.float32),
            pltpu.VMEM((C, HID), jnp.float32),
            pltpu.VMEM((C, HID), jnp.float32),
            pltpu.VMEM((16, HID), jnp.float32),
            pltpu.SemaphoreType.DMA,
            pltpu.SemaphoreType.DMA,
            pltpu.SemaphoreType.DMA,
        ])
    def k(src_h, dst_h, q_h, k_h, v_h, out_h, dout_h,
          acc, dacc, sidx, didx, lidx, lidx2, qr, kr, vr, denb, zb,
          sm0, sm1, sm2):
        cid = lax.axis_index("c")
        sid = lax.axis_index("s")
        zero = jnp.zeros((16,), jnp.float32)

        @pl.loop(0, HID, step=16)
        def _(j):
            @pl.loop(0, 16)
            def _(r):
                zb[r, pl.ds(j, 16)] = zero

        @pl.loop(0, C)
        def _(r):
            @pl.loop(0, HID, step=16)
            def _(j):
                denb[r, pl.ds(j, 16)] = zero

        row0 = sid * rows_pt

        @pl.loop(0, rows_pt, step=16)
        def _(t):
            pltpu.sync_copy(zb, acc.at[pl.ds(row0 + t, 16)])
        drow0 = sid * drows_pt

        @pl.loop(0, drows_pt, step=16)
        def _(t):
            pltpu.sync_copy(zb, dacc.at[pl.ds(drow0 + t, 16)])
        plsc.subcore_barrier()

        if split_dst:
            ebase = sid * per_tile
        else:
            ebase = (sid * 2 + cid) * per_tile
        lanes = jnp.arange(16, dtype=jnp.int32)

        @pl.loop(0, n_chunks)
        def _(ch):
            base = ebase + ch * C
            pltpu.sync_copy(src_h.at[pl.ds(base, C)], sidx)
            pltpu.sync_copy(dst_h.at[pl.ds(base, C)], didx)
            cq = pltpu.async_copy(q_h.at[didx], qr, sm0)
            ck = pltpu.async_copy(k_h.at[sidx], kr, sm1)
            cv = pltpu.async_copy(v_h.at[sidx], vr, sm2)
            hbase = cid * H

            @pl.loop(0, C, step=16)
            def _(g):
                d = didx[pl.ds(g, 16)]
                if split_dst:
                    loc = d - hbase
                    ok = (loc >= 0) & (loc < H)
                    loc = jnp.where(ok, loc, H)
                    loc2 = jnp.where(ok, loc, trash)
                else:
                    loc = d
                    loc2 = d
                lidx[pl.ds(g, 16)] = loc
                lidx2[pl.ds(g, 16)] = lax.shift_right_logical(loc2, 5)
            cq.wait()
            ck.wait()
            cv.wait()

            @pl.loop(0, C, step=16)
            def _(g):
                rows = lanes + g
                a = [zero, zero, zero, zero]
                for f in range(HID):
                    cf = jnp.full((16,), f, jnp.int32)
                    qv = plsc.load_gather(qr, [rows, cf])
                    kv = plsc.load_gather(kr, [rows, cf])
                    a[f // DH] = a[f // DH] + qv * kv
                e = [jnp.exp(x) for x in a]
                for f in range(HID):
                    cf = jnp.full((16,), f, jnp.int32)
                    vv = plsc.load_gather(vr, [rows, cf])
                    plsc.store_scatter(vr, [rows, cf], vv * e[f // DH])
                loc = lidx[pl.ds(g, 16)]
                dcol = lax.shift_left(loc & 31, 2)
                for h in range(NH):
                    plsc.store_scatter(denb, [rows, dcol + h], e[h])

            pltpu.sync_copy(vr, acc.at[lidx], add=True)
            pltpu.sync_copy(denb, dacc.at[lidx2], add=True)

            @pl.loop(0, C, step=16)
            def _(g):
                rows = lanes + g
                loc = lidx[pl.ds(g, 16)]
                dcol = lax.shift_left(loc & 31, 2)
                for h in range(NH):
                    plsc.store_scatter(denb, [rows, dcol + h], zero)

        plsc.subcore_barrier()
        pltpu.sync_copy(acc.at[pl.ds(row0, rows_pt)],
                        out_h.at[cid, pl.ds(row0, rows_pt)])
        pltpu.sync_copy(dacc.at[pl.ds(drow0, drows_pt)],
                        dout_h.at[cid, pl.ds(drow0, drows_pt)])

    return k(src, dst, q_tab, k_tab, v_tab)


def _sc_ctx(src, dst, w_tab, n_dst):
    E = src.shape[0]
    H = n_dst // 2
    Hp = _cdiv(H + 16, 128) * 128
    rows_pt = Hp // 16
    per_tile = E // 16
    n_chunks = per_tile // _CCTX
    mesh = plsc.VectorSubcoreMesh(core_axis_name="c", subcore_axis_name="s")

    @functools.partial(
        pl.kernel, mesh=mesh,
        compiler_params=pltpu.CompilerParams(needs_layout_passes=False),
        out_type=jax.ShapeDtypeStruct((2, Hp, HID), jnp.float32),
        scratch_types=[
            pltpu.VMEM_SHARED((Hp, HID), jnp.float32),
            pltpu.VMEM((_CCTX,), jnp.int32),
            pltpu.VMEM((_CCTX,), jnp.int32),
            pltpu.VMEM((_CCTX,), jnp.int32),
            pltpu.VMEM((_CCTX, HID), jnp.float32),
            pltpu.VMEM((16, HID), jnp.float32),
            pltpu.SemaphoreType.DMA,
        ])
    def k(src_h, dst_h, w_h, out_h, acc, sidx, didx, lidx, rows_b, zb, sm0):
        cid = lax.axis_index("c")
        sid = lax.axis_index("s")
        zero = jnp.zeros((16,), jnp.float32)

        @pl.loop(0, HID, step=16)
        def _(j):
            @pl.loop(0, 16)
            def _(r):
                zb[r, pl.ds(j, 16)] = zero

        row0 = sid * rows_pt

        @pl.loop(0, rows_pt, step=16)
        def _(t):
            pltpu.sync_copy(zb, acc.at[pl.ds(row0 + t, 16)])
        plsc.subcore_barrier()

        ebase = sid * per_tile
        hbase = cid * H

        @pl.loop(0, n_chunks)
        def _(ch):
            base = ebase + ch * _CCTX
            pltpu.sync_copy(src_h.at[pl.ds(base, _CCTX)], sidx)
            pltpu.sync_copy(dst_h.at[pl.ds(base, _CCTX)], didx)
            cg = pltpu.async_copy(w_h.at[sidx], rows_b, sm0)

            @pl.loop(0, _CCTX, step=16)
            def _(g):
                d = didx[pl.ds(g, 16)]
                loc = d - hbase
                ok = (loc >= 0) & (loc < H)
                lidx[pl.ds(g, 16)] = jnp.where(ok, loc, H)
            cg.wait()
            pltpu.sync_copy(rows_b, acc.at[lidx], add=True)

        plsc.subcore_barrier()
        pltpu.sync_copy(acc.at[pl.ds(row0, rows_pt)],
                        out_h.at[cid, pl.ds(row0, rows_pt)])

    return k(src, dst, w_tab)


def kernel(x_entity, x_passage, edge_index_e2p, edge_index_p2e, query_emb,
           params):
    p = params
    n_ent = x_entity.shape[0]
    n_psg = x_passage.shape[0]

    def bd(a):
        return jax.scipy.linalg.block_diag(*[a[h] for h in range(NH)])

    isq = 1.0 / math.sqrt(DH)
    sc_e = jnp.repeat(p['p_rel_p2e'], DH) * isq
    sc_p = jnp.repeat(p['p_rel_e2p'], DH) * isq

    q_e, kr_e2p, vr_e2p = _prep(
        x_entity, p['Wq_ent'].T, p['bq_ent'], sc_e, p['Wk_ent'].T,
        p['bk_ent'], bd(p['a_rel_e2p']), p['Wv_ent'].T, p['bv_ent'],
        bd(p['m_rel_e2p']))
    q_p, kr_p2e, vr_p2e = _prep(
        x_passage, p['Wq_psg'].T, p['bq_psg'], sc_p, p['Wk_psg'].T,
        p['bk_psg'], bd(p['a_rel_p2e']), p['Wv_psg'].T, p['bv_psg'],
        bd(p['m_rel_p2e']))

    s_ep, d_ep = edge_index_e2p[0], edge_index_e2p[1]
    s_pe, d_pe = edge_index_p2e[0], edge_index_p2e[1]
    hp = n_psg // 2
    acc_p, dacc_p = _sc_hgt(s_ep, d_ep, q_p, kr_e2p, vr_e2p, n_psg,
                            split_dst=True)
    num_p = jnp.concatenate([acc_p[0, :hp], acc_p[1, :hp]], axis=0)
    den_p = jnp.concatenate(
        [dacc_p[0].reshape(-1)[:hp * NH].reshape(hp, NH),
         dacc_p[1].reshape(-1)[:hp * NH].reshape(hp, NH)], axis=0)
    acc_e, dacc_e = _sc_hgt(s_pe, d_pe, q_e, kr_p2e, vr_p2e, n_ent,
                            split_dst=False)
    num_e = acc_e[0, :n_ent] + acc_e[1, :n_ent]
    den_e = (dacc_e[0] + dacc_e[1]).reshape(-1)[:n_ent * NH].reshape(
        n_ent, NH)
    den_p128 = jnp.repeat(den_p, DH, axis=-1)
    den_e128 = jnp.repeat(den_e, DH, axis=-1)

    h_ent, qkv = _ent_a(num_e, den_e128, x_entity, p['Wa_ent'].T,
                        p['ba_ent'], p['skip_ent'], p['in_proj_w'].T,
                        p['in_proj_b'])
    qh = qkv[:, :HID].reshape(n_ent, NH, DH).transpose(1, 0, 2)
    kh = qkv[:, HID:2 * HID].reshape(n_ent, NH, DH).transpose(1, 0, 2)
    vh = qkv[:, 2 * HID:].reshape(n_ent, NH, DH).transpose(1, 0, 2)
    g4 = _att(qh, kh, vh).transpose(1, 0, 2).reshape(n_ent, HID)
    weighted = _ent_c(h_ent, g4, p['out_proj_w'].T, p['out_proj_b'],
                      p['g_ent'], p['be_ent'], query_emb)

    acc_c = _sc_ctx(s_ep, d_ep, weighted, n_psg)
    ctx = jnp.concatenate(
        [acc_c[0, :n_psg // 2], acc_c[1, :n_psg // 2]], axis=0)

    W1aT = p['W1'][:, :HID].T
    W1bT = p['W1'][:, HID:].T
    W2T = jnp.zeros((HID, 8), jnp.float32).at[:, 0].set(p['W2'][0])
    b2 = jnp.zeros((8,), jnp.float32).at[0].set(p['b2'][0])
    scores = _psg(num_p, den_p128, x_passage, ctx, p['Wa_psg'].T,
                  p['ba_psg'], p['skip_psg'], p['g_psg'], p['be_psg'],
                  W1aT, W1bT, p['b1'], query_emb, W2T, b2)
    return scores[:, 0]
